# Initial kernel scaffold; baseline (speedup 1.0000x reference)
import jax
import jax.numpy as jnp
from jax.experimental import pallas as pl


def kernel(x, g, b, W_qkv, W_out, gamma):
    raise NotImplementedError("write your pallas kernel here")


# speedup vs baseline: 5.3293x; 5.3293x over previous
import jax
import jax.numpy as jnp
from jax.experimental import pallas as pl

_DIM = 384
_DIM_HEAD = 64
_HEADS = 8
_TOPK = 16
_H = 32
_W = 32
_P = _H * _W
_INNER = _DIM_HEAD * _HEADS


def _topk_select(score, jj, ii, tri):
    f32 = jnp.float32
    A = jnp.broadcast_to(score, (32, 32))
    At = jnp.transpose(A)
    beats = (A > At) | ((A == At) & (jj < ii))
    rank = jnp.sum(beats.astype(f32), axis=1, keepdims=True)
    maskf = (rank < float(_TOPK)).astype(f32)
    maskT = jnp.transpose(maskf)
    pos = jnp.sum(tri * maskT, axis=1, keepdims=True)
    posT = jnp.transpose(pos)
    slots = jax.lax.broadcasted_iota(jnp.int32, (_TOPK, 32), 0).astype(f32)
    S = (slots == jnp.broadcast_to(posT, (_TOPK, 32))).astype(f32)
    S = S * jnp.broadcast_to(maskT, (_TOPK, 32))
    return S


def _dpsa_body(x_ref, g_ref, b_ref, wqkv_ref, wout_ref, gamma_ref, y_ref):
    f32 = jnp.float32
    xb = x_ref[0]
    mean = jnp.mean(xb, axis=0, keepdims=True)
    xc = xb - mean
    var = jnp.mean(xc * xc, axis=0, keepdims=True)
    xn = xc * jax.lax.rsqrt(var + 1e-5) * g_ref[...] + b_ref[...]
    qkv = jnp.dot(wqkv_ref[...], xn, preferred_element_type=f32)

    ph = jax.lax.broadcasted_iota(jnp.int32, (_P, _H), 0) // _W
    ch = jax.lax.broadcasted_iota(jnp.int32, (_P, _H), 1)
    er = (ph == ch).astype(f32)
    pw = jax.lax.broadcasted_iota(jnp.int32, (_P, _W), 0) % _W
    cw = jax.lax.broadcasted_iota(jnp.int32, (_P, _W), 1)
    ew = (pw == cw).astype(f32)
    nk = _TOPK * _TOPK
    ech = (jax.lax.broadcasted_iota(jnp.int32, (_TOPK, nk), 0)
           == jax.lax.broadcasted_iota(jnp.int32, (_TOPK, nk), 1) // _TOPK
           ).astype(f32)
    ecw = (jax.lax.broadcasted_iota(jnp.int32, (_TOPK, nk), 0)
           == jax.lax.broadcasted_iota(jnp.int32, (_TOPK, nk), 1) % _TOPK
           ).astype(f32)
    ii = jax.lax.broadcasted_iota(jnp.int32, (32, 32), 0)
    jj = jax.lax.broadcasted_iota(jnp.int32, (32, 32), 1)
    tri = (jj < ii).astype(f32)

    outs = []
    for h in range(_HEADS):
        qh = qkv[h * _DIM_HEAD:(h + 1) * _DIM_HEAD]
        kh = qkv[_INNER + h * _DIM_HEAD:_INNER + (h + 1) * _DIM_HEAD]
        vh = qkv[2 * _INNER + h * _DIM_HEAD:2 * _INNER + (h + 1) * _DIM_HEAD]
        qn = qh * (1.0 / jnp.maximum(
            jnp.sqrt(jnp.sum(qh * qh, axis=0, keepdims=True)), 1e-12))
        kn = kh * (1.0 / jnp.maximum(
            jnp.sqrt(jnp.sum(kh * kh, axis=0, keepdims=True)), 1e-12))
        k_abs = jnp.abs(kn)
        q_probe = jnp.sum(jnp.abs(qn), axis=1, keepdims=True)
        t = jnp.sum(q_probe * k_abs, axis=0, keepdims=True)
        score_r = jnp.dot(t, er, preferred_element_type=f32)
        score_c = jnp.dot(t, ew, preferred_element_type=f32)
        Sh = _topk_select(score_r, jj, ii, tri)
        Sw = _topk_select(score_c, jj, ii, tri)
        A0 = jnp.dot(er, jnp.transpose(Sh), preferred_element_type=f32)
        Aq = jnp.dot(A0, ech, preferred_element_type=f32)
        B0 = jnp.dot(ew, jnp.transpose(Sw), preferred_element_type=f32)
        Bq = jnp.dot(B0, ecw, preferred_element_type=f32)
        Qm = Aq * Bq
        kp = jnp.dot(kn, Qm, preferred_element_type=f32)
        vp = jnp.dot(vh, Qm, preferred_element_type=f32)
        sim = jax.lax.dot_general(qn, kp, (((0,), (0,)), ((), ())),
                                  preferred_element_type=f32)
        m = jnp.max(sim, axis=1, keepdims=True)
        e = jnp.exp(sim - m)
        attn = e / jnp.sum(e, axis=1, keepdims=True)
        oh = jax.lax.dot_general(attn, vp, (((1,), (1,)), ((), ())),
                                 preferred_element_type=f32)
        outs.append(oh)
    o = jnp.concatenate(outs, axis=1)
    out = jax.lax.dot_general(wout_ref[...], o, (((1,), (1,)), ((), ())),
                              preferred_element_type=f32)
    y_ref[0] = gamma_ref[0, 0] * out + xn


def kernel(x, g, b, W_qkv, W_out, gamma):
    B, C, H, W = x.shape
    x2 = x.reshape(B, C, H * W)
    g2 = g.reshape(C, 1)
    b2 = b.reshape(C, 1)
    gm = jnp.asarray(gamma, jnp.float32).reshape(1, 1)
    y2 = pl.pallas_call(
        _dpsa_body,
        grid=(B,),
        in_specs=[
            pl.BlockSpec((1, C, H * W), lambda i: (i, 0, 0)),
            pl.BlockSpec((C, 1), lambda i: (0, 0)),
            pl.BlockSpec((C, 1), lambda i: (0, 0)),
            pl.BlockSpec(W_qkv.shape, lambda i: (0, 0)),
            pl.BlockSpec(W_out.shape, lambda i: (0, 0)),
            pl.BlockSpec((1, 1), lambda i: (0, 0)),
        ],
        out_specs=pl.BlockSpec((1, C, H * W), lambda i: (i, 0, 0)),
        out_shape=jax.ShapeDtypeStruct((B, C, H * W), jnp.float32),
    )(x2, g2, b2, W_qkv, W_out, gm)
    return y2.reshape(B, C, H, W)
